# baseline (speedup 1.0000x reference)
import functools

import jax
import jax.numpy as jnp
import numpy as np
from jax import lax
from jax.experimental import pallas as pl
from jax.experimental.pallas import tpu as pltpu
from jax.experimental.pallas import tpu_sc as plsc

M = 1024
N = 8192
K = N // 2 - 1
BLOCK_M = 128

NW = 32
ROWS_PER_W = M // NW
VREGS = N // 16

_SIGN = np.int32(np.uint32(0x80000000))
_LOW31 = np.int32(0x7FFFFFFF)


_MIN32 = np.int32(np.uint32(0x80000000))


def _flag(n):
    return jnp.where(n >= K, 1, 0)


def _decide(p_u, nh, shift, n1, n2, n3):
    bits = _flag(n1) + _flag(n2) + _flag(n3)
    p_new = p_u | lax.shift_left(bits, shift)
    nh_new = jnp.where(
        bits == 0, n1, jnp.where(bits == 1, n2, jnp.where(bits == 2, n3, nh))
    )
    return p_new, nh_new


def _cand_vecs(p_u, shift):
    c1 = (p_u | lax.shift_left(jnp.int32(1), shift)) ^ _SIGN
    c2 = (p_u | lax.shift_left(jnp.int32(2), shift)) ^ _SIGN
    c3 = (p_u | lax.shift_left(jnp.int32(3), shift)) ^ _SIGN
    return (
        jnp.full((16,), c1, jnp.int32),
        jnp.full((16,), c2, jnp.int32),
        jnp.full((16,), c3, jnp.int32),
    )


def _count3(x, accs, c1v, c2v, c3v):
    a1, a2, a3 = accs
    a1 = a1 + jnp.where(x >= c1v, 1, 0)
    a2 = a2 + jnp.where(x >= c2v, 1, 0)
    a3 = a3 + jnp.where(x >= c3v, 1, 0)
    return (a1, a2, a3)


def _compact_pass(src, dst, nv_src, p_u, shift_prev):
    lo_v = jnp.full((16,), p_u ^ _SIGN, jnp.int32)
    hi_u = p_u + lax.shift_left(jnp.int32(1), shift_prev)
    hi_v = jnp.full((16,), hi_u ^ _SIGN, jnp.int32)
    hz_v = jnp.full((16,), hi_u, jnp.int32) == 0
    z = jnp.zeros((16,), jnp.int32)
    lane = lax.iota(jnp.int32, 16)
    idx15 = jnp.full((16,), 15, jnp.int32)

    def body(j, off_v):
        for k in range(4):
            x = src[pl.ds((j * 4 + k) * 16, 16)]
            m_in = (x >= lo_v) & ((x < hi_v) | hz_v)
            cum = plsc.cumsum(jnp.where(m_in, 1, 0))
            plsc.store_scatter(dst, [off_v + (cum - 1)], x, mask=m_in)
            off_v = off_v + cum.at[idx15].get(mode="promise_in_bounds")
        return off_v

    n4 = (nv_src + 3) >> 2
    off_v = lax.fori_loop(0, n4, body, z)
    off = off_v[0]
    minv = jnp.full((16,), _MIN32, jnp.int32)
    ones = lane >= 0
    for k in range(4):
        plsc.store_scatter(dst, [off + k * 16 + lane], minv, mask=ones)
    return (off + 15) >> 4


def _compact_count_pass(src, dst, nv_src, p_u, shift_prev, cands):
    c1v, c2v, c3v = cands
    lo_v = jnp.full((16,), p_u ^ _SIGN, jnp.int32)
    hi_u = p_u + lax.shift_left(jnp.int32(1), shift_prev)
    hi_v = jnp.full((16,), hi_u ^ _SIGN, jnp.int32)
    hz_v = jnp.full((16,), hi_u, jnp.int32) == 0
    z = jnp.zeros((16,), jnp.int32)
    lane = lax.iota(jnp.int32, 16)

    idx15 = jnp.full((16,), 15, jnp.int32)

    def body(j, carry):
        a1, a2, a3, off_v = carry
        xs, cums = [], []
        for k in range(4):
            x = src[pl.ds((j * 4 + k) * 16, 16)]
            m_in = (x >= lo_v) & ((x < hi_v) | hz_v)
            xs.append((x, m_in))
            cums.append(plsc.cumsum(jnp.where(m_in, 1, 0)))
        for k in range(4):
            x, m_in = xs[k]
            plsc.store_scatter(dst, [off_v + (cums[k] - 1)], x, mask=m_in)
            off_v = off_v + cums[k].at[idx15].get(mode="promise_in_bounds")
            a1, a2, a3 = _count3(x, (a1, a2, a3), c1v, c2v, c3v)
        return (a1, a2, a3, off_v)

    n4 = (nv_src + 3) >> 2
    a1, a2, a3, off_v = lax.fori_loop(0, n4, body, (z, z, z, z))
    off = off_v[0]
    minv = jnp.full((16,), _MIN32, jnp.int32)
    ones = lane >= 0
    for k in range(4):
        plsc.store_scatter(dst, [off + k * 16 + lane], minv, mask=ones)
    return jnp.sum(a1), jnp.sum(a2), jnp.sum(a3), (off + 15) >> 4


def _count_pass(src, nv_src, cands):
    c1v, c2v, c3v = cands
    z = jnp.zeros((16,), jnp.int32)

    def body(j, accs):
        for k in range(4):
            x = src[pl.ds((j * 4 + k) * 16, 16)]
            accs = _count3(x, accs, c1v, c2v, c3v)
        return accs

    n4 = (nv_src + 3) >> 2
    a1, a2, a3 = lax.fori_loop(0, n4, body, (z, z, z))
    return jnp.sum(a1), jnp.sum(a2), jnp.sum(a3)


def _sc_select_body(
    inputs_hbm, targets_hbm, out_hbm, row_v, key_v, buf_b, buf_c, tgt_v, out_v
):
    wid = lax.axis_index("s") * 2 + lax.axis_index("c")
    base = wid * ROWS_PER_W
    pltpu.sync_copy(targets_hbm.at[pl.ds(base, ROWS_PER_W)], tgt_v)
    lane = lax.iota(jnp.int32, 16)
    l0 = lane == 0

    def row_body(r, carry):
        pltpu.sync_copy(inputs_hbm.at[base + r], row_v)
        tb = plsc.load_gather(tgt_v, [jnp.full((16,), r, jnp.int32)])

        cands0 = _cand_vecs(jnp.int32(0), 30)
        c1v, c2v, c3v = cands0
        z = jnp.zeros((16,), jnp.int32)

        def key_body(j, accs):
            for k in range(4):
                jj = j * 4 + k
                x = row_v[pl.ds(jj * 16, 16)]
                b = plsc.bitcast(x, jnp.int32)
                s = jnp.where(b >= 0, b, b ^ _LOW31)
                key_v[pl.ds(jj * 16, 16)] = s
                accs = _count3(s, accs, c1v, c2v, c3v)
            return accs

        a1, a2, a3 = lax.fori_loop(0, VREGS // 4, key_body, (z, z, z))
        s_pos = plsc.load_gather(key_v, [tb])[0]
        plsc.store_scatter(
            key_v, [tb], jnp.full((16,), _MIN32, jnp.int32), mask=l0
        )
        n1 = jnp.sum(a1) - jnp.where(s_pos >= c1v[0], 1, 0)
        n2 = jnp.sum(a2) - jnp.where(s_pos >= c2v[0], 1, 0)
        n3 = jnp.sum(a3) - jnp.where(s_pos >= c3v[0], 1, 0)
        p_u, nh0 = _decide(jnp.int32(0), jnp.int32(0), jnp.int32(30), n1, n2, n3)

        nv_b = _compact_pass(key_v, buf_b, jnp.int32(VREGS), p_u, jnp.int32(30))
        n1, n2, n3 = _count_pass(buf_b, nv_b, _cand_vecs(p_u, 28))
        p_u, nh1 = _decide(p_u, nh0, jnp.int32(28), n1 + nh0, n2 + nh0, n3 + nh0)

        n1, n2, n3, nv_c = _compact_count_pass(
            buf_b, buf_c, nv_b, p_u, jnp.int32(28), _cand_vecs(p_u, 26)
        )
        p_u, nh2 = _decide(
            p_u, nh1, jnp.int32(26), n1 + nh0, n2 + nh0, n3 + nh0
        )

        n1, n2, n3, nv_d = _compact_count_pass(
            buf_c, buf_b, nv_c, p_u, jnp.int32(26), _cand_vecs(p_u, 24)
        )
        p_u, nh3 = _decide(
            p_u, nh2, jnp.int32(24), n1 + nh1, n2 + nh1, n3 + nh1
        )

        def pass_body(q, p_u):
            shift = 30 - 2 * q
            n1, n2, n3 = _count_pass(buf_b, nv_d, _cand_vecs(p_u, shift))
            p_new, _ = _decide(
                p_u, jnp.int32(0), shift, n1 + nh2, n2 + nh2, n3 + nh2
            )
            return p_new

        p_u = lax.fori_loop(4, 16, pass_body, p_u)
        plsc.store_scatter(out_v, [jnp.full((16,), r, jnp.int32)],
                           jnp.full((16,), p_u, jnp.int32), mask=l0)
        return carry

    lax.fori_loop(0, ROWS_PER_W, row_body, 0)
    pltpu.sync_copy(out_v, out_hbm.at[pl.ds(base, ROWS_PER_W)])


_sc_select = functools.partial(
    pl.kernel,
    out_type=jax.ShapeDtypeStruct((M,), jnp.int32),
    mesh=plsc.VectorSubcoreMesh(core_axis_name="c", subcore_axis_name="s"),
    scratch_types=[
        pltpu.VMEM((N,), jnp.float32),
        pltpu.VMEM((N,), jnp.int32),
        pltpu.VMEM((N + 128,), jnp.int32),
        pltpu.VMEM((N + 128,), jnp.int32),
        pltpu.VMEM((ROWS_PER_W,), jnp.int32),
        pltpu.VMEM((ROWS_PER_W,), jnp.int32),
    ],
    compiler_params=pltpu.CompilerParams(needs_layout_passes=False),
)(_sc_select_body)


def _softplus(x):
    return jnp.maximum(x, 0.0) + jnp.log1p(jnp.exp(-jnp.abs(x)))


def _finalize_body(x_ref, tgt_ref, tu_ref, out_ref):
    pid = pl.program_id(0)
    x = x_ref[...]
    tgt = tgt_ref[pl.ds(pid * BLOCK_M, BLOCK_M), :]
    t_u = tu_ref[pl.ds(pid * BLOCK_M, BLOCK_M), :]

    col = jax.lax.broadcasted_iota(jnp.int32, (BLOCK_M, N), 1)
    pos_mask = col == tgt
    neg_mask = jnp.logical_not(pos_mask)

    bits = jax.lax.bitcast_convert_type(x, jnp.int32)
    s = jnp.where(bits >= 0, bits, bits ^ _LOW31)

    t_s = t_u ^ _SIGN
    t_bits = jnp.where(t_s >= 0, t_s, t_s ^ _LOW31)
    t_f = jax.lax.bitcast_convert_type(t_bits, jnp.float32)

    gt = (s > t_s) & neg_mask
    c = jnp.sum(gt.astype(jnp.int32), axis=1, keepdims=True).astype(jnp.float32)
    sp = _softplus(x)
    sum_sp = jnp.sum(jnp.where(gt, sp, 0.0), axis=1, keepdims=True)
    l_neg = (sum_sp + (K - c) * _softplus(t_f)) * (1.0 / K)

    pos = jnp.sum(jnp.where(pos_mask, x, 0.0), axis=1, keepdims=True)
    per_row = _softplus(-pos) + l_neg

    @pl.when(pid == 0)
    def _():
        out_ref[...] = jnp.zeros((1, 1), jnp.float32)

    out_ref[...] += jnp.sum(per_row).reshape(1, 1) * (1.0 / M)


@jax.jit
def kernel(inputs, targets):
    tgt = targets.astype(jnp.int32)
    t_u = _sc_select(inputs, tgt)
    grid = M // BLOCK_M
    out = pl.pallas_call(
        _finalize_body,
        grid=(grid,),
        in_specs=[
            pl.BlockSpec((BLOCK_M, N), lambda i: (i, 0)),
            pl.BlockSpec((M, 1), lambda i: (0, 0)),
            pl.BlockSpec((M, 1), lambda i: (0, 0)),
        ],
        out_specs=pl.BlockSpec((1, 1), lambda i: (0, 0)),
        out_shape=jax.ShapeDtypeStruct((1, 1), jnp.float32),
        compiler_params=pltpu.CompilerParams(
            dimension_semantics=("arbitrary",),
        ),
    )(inputs, tgt.reshape(M, 1), t_u.reshape(M, 1))
    return out[0, 0]


# speedup vs baseline: 1.4085x; 1.4085x over previous
import functools

import jax
import jax.numpy as jnp
import numpy as np
from jax import lax
from jax.experimental import pallas as pl
from jax.experimental.pallas import tpu as pltpu
from jax.experimental.pallas import tpu_sc as plsc

M = 1024
N = 8192
K = N // 2 - 1
BLOCK_M = 128

NW = 32
ROWS_PER_W = M // NW
VREGS = N // 16

_SIGN = np.int32(np.uint32(0x80000000))
_LOW31 = np.int32(0x7FFFFFFF)


_MIN32 = np.int32(np.uint32(0x80000000))


def _flag(n):
    return jnp.where(n >= K, 1, 0)


def _decide(p_u, nh, shift, n1, n2, n3):
    bits = _flag(n1) + _flag(n2) + _flag(n3)
    p_new = p_u | lax.shift_left(bits, shift)
    nh_new = jnp.where(
        bits == 0, n1, jnp.where(bits == 1, n2, jnp.where(bits == 2, n3, nh))
    )
    return p_new, nh_new


def _cand_vecs(p_u, shift):
    c1 = (p_u | lax.shift_left(jnp.int32(1), shift)) ^ _SIGN
    c2 = (p_u | lax.shift_left(jnp.int32(2), shift)) ^ _SIGN
    c3 = (p_u | lax.shift_left(jnp.int32(3), shift)) ^ _SIGN
    return (
        jnp.full((16,), c1, jnp.int32),
        jnp.full((16,), c2, jnp.int32),
        jnp.full((16,), c3, jnp.int32),
    )


def _count3(x, accs, c1v, c2v, c3v):
    a1, a2, a3 = accs
    a1 = a1 + jnp.where(x >= c1v, 1, 0)
    a2 = a2 + jnp.where(x >= c2v, 1, 0)
    a3 = a3 + jnp.where(x >= c3v, 1, 0)
    return (a1, a2, a3)


def _compact_pass(src, dst, nv_src, p_u, shift_prev):
    lo_v = jnp.full((16,), p_u ^ _SIGN, jnp.int32)
    hi_u = p_u + lax.shift_left(jnp.int32(1), shift_prev)
    hi_v = jnp.full((16,), hi_u ^ _SIGN, jnp.int32)
    hz_v = jnp.full((16,), hi_u, jnp.int32) == 0
    z = jnp.zeros((16,), jnp.int32)
    lane = lax.iota(jnp.int32, 16)
    idx15 = jnp.full((16,), 15, jnp.int32)

    def body(j, off_v):
        for k in range(4):
            x = src[pl.ds((j * 4 + k) * 16, 16)]
            m_in = (x >= lo_v) & ((x < hi_v) | hz_v)
            cum = plsc.cumsum(jnp.where(m_in, 1, 0))
            plsc.store_scatter(dst, [off_v + (cum - 1)], x, mask=m_in)
            off_v = off_v + cum.at[idx15].get(mode="promise_in_bounds")
        return off_v

    n4 = (nv_src + 3) >> 2
    off_v = lax.fori_loop(0, n4, body, z)
    off = off_v[0]
    minv = jnp.full((16,), _MIN32, jnp.int32)
    ones = lane >= 0
    for k in range(4):
        plsc.store_scatter(dst, [off + k * 16 + lane], minv, mask=ones)
    return (off + 15) >> 4


def _compact_count_pass(src, dst, nv_src, p_u, shift_prev, cands):
    c1v, c2v, c3v = cands
    lo_v = jnp.full((16,), p_u ^ _SIGN, jnp.int32)
    hi_u = p_u + lax.shift_left(jnp.int32(1), shift_prev)
    hi_v = jnp.full((16,), hi_u ^ _SIGN, jnp.int32)
    hz_v = jnp.full((16,), hi_u, jnp.int32) == 0
    z = jnp.zeros((16,), jnp.int32)
    lane = lax.iota(jnp.int32, 16)

    idx15 = jnp.full((16,), 15, jnp.int32)

    def body(j, carry):
        a1, a2, a3, off_v = carry
        xs, cums = [], []
        for k in range(4):
            x = src[pl.ds((j * 4 + k) * 16, 16)]
            m_in = (x >= lo_v) & ((x < hi_v) | hz_v)
            xs.append((x, m_in))
            cums.append(plsc.cumsum(jnp.where(m_in, 1, 0)))
        for k in range(4):
            x, m_in = xs[k]
            plsc.store_scatter(dst, [off_v + (cums[k] - 1)], x, mask=m_in)
            off_v = off_v + cums[k].at[idx15].get(mode="promise_in_bounds")
            a1, a2, a3 = _count3(x, (a1, a2, a3), c1v, c2v, c3v)
        return (a1, a2, a3, off_v)

    n4 = (nv_src + 3) >> 2
    a1, a2, a3, off_v = lax.fori_loop(0, n4, body, (z, z, z, z))
    off = off_v[0]
    minv = jnp.full((16,), _MIN32, jnp.int32)
    ones = lane >= 0
    for k in range(4):
        plsc.store_scatter(dst, [off + k * 16 + lane], minv, mask=ones)
    return jnp.sum(a1), jnp.sum(a2), jnp.sum(a3), (off + 15) >> 4


def _count_pass(src, nv_src, cands):
    c1v, c2v, c3v = cands
    z = jnp.zeros((16,), jnp.int32)

    def body(j, accs):
        for k in range(4):
            x = src[pl.ds((j * 4 + k) * 16, 16)]
            accs = _count3(x, accs, c1v, c2v, c3v)
        return accs

    n4 = (nv_src + 3) >> 2
    a1, a2, a3 = lax.fori_loop(0, n4, body, (z, z, z))
    return jnp.sum(a1), jnp.sum(a2), jnp.sum(a3)


def _sc_select_body(
    inputs_hbm, targets_hbm, out_hbm, row_v, key_v, buf_b, buf_c, tgt_v, out_v
):
    wid = lax.axis_index("s") * 2 + lax.axis_index("c")
    base = wid * ROWS_PER_W
    pltpu.sync_copy(targets_hbm.at[pl.ds(base, ROWS_PER_W)], tgt_v)
    lane = lax.iota(jnp.int32, 16)
    l0 = lane == 0

    def row_body(r, carry):
        pltpu.sync_copy(inputs_hbm.at[base + r], row_v)
        tb = plsc.load_gather(tgt_v, [jnp.full((16,), r, jnp.int32)])

        cands0 = _cand_vecs(jnp.int32(0), 30)
        c1v, c2v, c3v = cands0
        z = jnp.zeros((16,), jnp.int32)

        def key_body(j, accs):
            for k in range(4):
                jj = j * 4 + k
                x = row_v[pl.ds(jj * 16, 16)]
                b = plsc.bitcast(x, jnp.int32)
                s = jnp.where(b >= 0, b, b ^ _LOW31)
                key_v[pl.ds(jj * 16, 16)] = s
                accs = _count3(s, accs, c1v, c2v, c3v)
            return accs

        a1, a2, a3 = lax.fori_loop(0, VREGS // 4, key_body, (z, z, z))
        s_pos = plsc.load_gather(key_v, [tb])[0]
        plsc.store_scatter(
            key_v, [tb], jnp.full((16,), _MIN32, jnp.int32), mask=l0
        )
        n1 = jnp.sum(a1) - jnp.where(s_pos >= c1v[0], 1, 0)
        n2 = jnp.sum(a2) - jnp.where(s_pos >= c2v[0], 1, 0)
        n3 = jnp.sum(a3) - jnp.where(s_pos >= c3v[0], 1, 0)
        p_u, nh0 = _decide(jnp.int32(0), jnp.int32(0), jnp.int32(30), n1, n2, n3)

        n1, n2, n3, nv_b = _compact_count_pass(
            key_v, buf_b, jnp.int32(VREGS), p_u, jnp.int32(30), _cand_vecs(p_u, 28)
        )
        p_u, nh1 = _decide(p_u, nh0, jnp.int32(28), n1, n2, n3)

        n1, n2, n3, nv_c = _compact_count_pass(
            buf_b, buf_c, nv_b, p_u, jnp.int32(28), _cand_vecs(p_u, 26)
        )
        p_u, nh2 = _decide(
            p_u, nh1, jnp.int32(26), n1 + nh0, n2 + nh0, n3 + nh0
        )

        n1, n2, n3, nv_d = _compact_count_pass(
            buf_c, buf_b, nv_c, p_u, jnp.int32(26), _cand_vecs(p_u, 24)
        )
        p_u, nh3 = _decide(
            p_u, nh2, jnp.int32(24), n1 + nh1, n2 + nh1, n3 + nh1
        )

        def pass_body(q, p_u):
            shift = 30 - 2 * q
            n1, n2, n3 = _count_pass(buf_b, nv_d, _cand_vecs(p_u, shift))
            p_new, _ = _decide(
                p_u, jnp.int32(0), shift, n1 + nh2, n2 + nh2, n3 + nh2
            )
            return p_new

        p_u = lax.fori_loop(4, 16, pass_body, p_u)
        plsc.store_scatter(out_v, [jnp.full((16,), r, jnp.int32)],
                           jnp.full((16,), p_u, jnp.int32), mask=l0)
        return carry

    lax.fori_loop(0, ROWS_PER_W, row_body, 0)
    pltpu.sync_copy(out_v, out_hbm.at[pl.ds(base, ROWS_PER_W)])


_sc_select = functools.partial(
    pl.kernel,
    out_type=jax.ShapeDtypeStruct((M,), jnp.int32),
    mesh=plsc.VectorSubcoreMesh(core_axis_name="c", subcore_axis_name="s"),
    scratch_types=[
        pltpu.VMEM((N,), jnp.float32),
        pltpu.VMEM((N,), jnp.int32),
        pltpu.VMEM((N + 128,), jnp.int32),
        pltpu.VMEM((N + 128,), jnp.int32),
        pltpu.VMEM((ROWS_PER_W,), jnp.int32),
        pltpu.VMEM((ROWS_PER_W,), jnp.int32),
    ],
    compiler_params=pltpu.CompilerParams(needs_layout_passes=False),
)(_sc_select_body)


def _softplus(x):
    return jnp.maximum(x, 0.0) + jnp.log1p(jnp.exp(-jnp.abs(x)))


def _finalize_body(x_ref, tgt_ref, tu_ref, out_ref):
    pid = pl.program_id(0)
    x = x_ref[...]
    tgt = tgt_ref[pl.ds(pid * BLOCK_M, BLOCK_M), :]
    t_u = tu_ref[pl.ds(pid * BLOCK_M, BLOCK_M), :]

    col = jax.lax.broadcasted_iota(jnp.int32, (BLOCK_M, N), 1)
    pos_mask = col == tgt
    neg_mask = jnp.logical_not(pos_mask)

    bits = jax.lax.bitcast_convert_type(x, jnp.int32)
    s = jnp.where(bits >= 0, bits, bits ^ _LOW31)

    t_s = t_u ^ _SIGN
    t_bits = jnp.where(t_s >= 0, t_s, t_s ^ _LOW31)
    t_f = jax.lax.bitcast_convert_type(t_bits, jnp.float32)

    gt = (s > t_s) & neg_mask
    c = jnp.sum(gt.astype(jnp.int32), axis=1, keepdims=True).astype(jnp.float32)
    sp = _softplus(x)
    sum_sp = jnp.sum(jnp.where(gt, sp, 0.0), axis=1, keepdims=True)
    l_neg = (sum_sp + (K - c) * _softplus(t_f)) * (1.0 / K)

    pos = jnp.sum(jnp.where(pos_mask, x, 0.0), axis=1, keepdims=True)
    per_row = _softplus(-pos) + l_neg

    @pl.when(pid == 0)
    def _():
        out_ref[...] = jnp.zeros((1, 1), jnp.float32)

    out_ref[...] += jnp.sum(per_row).reshape(1, 1) * (1.0 / M)


@jax.jit
def kernel(inputs, targets):
    tgt = targets.astype(jnp.int32)
    t_u = _sc_select(inputs, tgt)
    grid = M // BLOCK_M
    out = pl.pallas_call(
        _finalize_body,
        grid=(grid,),
        in_specs=[
            pl.BlockSpec((BLOCK_M, N), lambda i: (i, 0)),
            pl.BlockSpec((M, 1), lambda i: (0, 0)),
            pl.BlockSpec((M, 1), lambda i: (0, 0)),
        ],
        out_specs=pl.BlockSpec((1, 1), lambda i: (0, 0)),
        out_shape=jax.ShapeDtypeStruct((1, 1), jnp.float32),
        compiler_params=pltpu.CompilerParams(
            dimension_semantics=("arbitrary",),
        ),
    )(inputs, tgt.reshape(M, 1), t_u.reshape(M, 1))
    return out[0, 0]


# speedup vs baseline: 1.5164x; 1.0766x over previous
import functools

import jax
import jax.numpy as jnp
import numpy as np
from jax import lax
from jax.experimental import pallas as pl
from jax.experimental.pallas import tpu as pltpu
from jax.experimental.pallas import tpu_sc as plsc

M = 1024
N = 8192
K = N // 2 - 1
BLOCK_M = 128

NW = 32
ROWS_PER_W = M // NW
VREGS = N // 16

_SIGN = np.int32(np.uint32(0x80000000))
_LOW31 = np.int32(0x7FFFFFFF)


_MIN32 = np.int32(np.uint32(0x80000000))


def _flag(n):
    return jnp.where(n >= K, 1, 0)


def _decide(p_u, nh, shift, n1, n2, n3):
    bits = _flag(n1) + _flag(n2) + _flag(n3)
    p_new = p_u | lax.shift_left(bits, shift)
    nh_new = jnp.where(
        bits == 0, n1, jnp.where(bits == 1, n2, jnp.where(bits == 2, n3, nh))
    )
    return p_new, nh_new


def _cand_vecs(p_u, shift):
    c1 = (p_u | lax.shift_left(jnp.int32(1), shift)) ^ _SIGN
    c2 = (p_u | lax.shift_left(jnp.int32(2), shift)) ^ _SIGN
    c3 = (p_u | lax.shift_left(jnp.int32(3), shift)) ^ _SIGN
    return (
        jnp.full((16,), c1, jnp.int32),
        jnp.full((16,), c2, jnp.int32),
        jnp.full((16,), c3, jnp.int32),
    )


def _count3(x, accs, c1v, c2v, c3v):
    a1, a2, a3 = accs
    a1 = a1 + jnp.where(x >= c1v, 1, 0)
    a2 = a2 + jnp.where(x >= c2v, 1, 0)
    a3 = a3 + jnp.where(x >= c3v, 1, 0)
    return (a1, a2, a3)


def _compact_pass(src, dst, nv_src, p_u, shift_prev):
    lo_v = jnp.full((16,), p_u ^ _SIGN, jnp.int32)
    hi_u = p_u + lax.shift_left(jnp.int32(1), shift_prev)
    hi_v = jnp.full((16,), hi_u ^ _SIGN, jnp.int32)
    hz_v = jnp.full((16,), hi_u, jnp.int32) == 0
    z = jnp.zeros((16,), jnp.int32)
    lane = lax.iota(jnp.int32, 16)
    idx15 = jnp.full((16,), 15, jnp.int32)

    def body(j, off_v):
        for k in range(4):
            x = src[pl.ds((j * 4 + k) * 16, 16)]
            m_in = (x >= lo_v) & ((x < hi_v) | hz_v)
            cum = plsc.cumsum(jnp.where(m_in, 1, 0))
            plsc.store_scatter(dst, [off_v + (cum - 1)], x, mask=m_in)
            off_v = off_v + cum.at[idx15].get(mode="promise_in_bounds")
        return off_v

    n4 = (nv_src + 3) >> 2
    off_v = lax.fori_loop(0, n4, body, z)
    off = off_v[0]
    minv = jnp.full((16,), _MIN32, jnp.int32)
    ones = lane >= 0
    for k in range(4):
        plsc.store_scatter(dst, [off + k * 16 + lane], minv, mask=ones)
    return (off + 15) >> 4


def _compact_count_pass(src, dst, nv_src, p_u, shift_prev, cands):
    c1v, c2v, c3v = cands
    lo_v = jnp.full((16,), p_u ^ _SIGN, jnp.int32)
    hi_u = p_u + lax.shift_left(jnp.int32(1), shift_prev)
    hi_v = jnp.full((16,), hi_u ^ _SIGN, jnp.int32)
    hz_v = jnp.full((16,), hi_u, jnp.int32) == 0
    z = jnp.zeros((16,), jnp.int32)
    lane = lax.iota(jnp.int32, 16)

    idx15 = jnp.full((16,), 15, jnp.int32)

    def body(j, carry):
        a1, a2, a3, off_v = carry
        xs, cums = [], []
        for k in range(4):
            x = src[pl.ds((j * 4 + k) * 16, 16)]
            m_in = (x >= lo_v) & ((x < hi_v) | hz_v)
            xs.append((x, m_in))
            cums.append(plsc.cumsum(jnp.where(m_in, 1, 0)))
        for k in range(4):
            x, m_in = xs[k]
            plsc.store_scatter(dst, [off_v + (cums[k] - 1)], x, mask=m_in)
            off_v = off_v + cums[k].at[idx15].get(mode="promise_in_bounds")
            a1, a2, a3 = _count3(x, (a1, a2, a3), c1v, c2v, c3v)
        return (a1, a2, a3, off_v)

    n4 = (nv_src + 3) >> 2
    a1, a2, a3, off_v = lax.fori_loop(0, n4, body, (z, z, z, z))
    off = off_v[0]
    minv = jnp.full((16,), _MIN32, jnp.int32)
    ones = lane >= 0
    for k in range(4):
        plsc.store_scatter(dst, [off + k * 16 + lane], minv, mask=ones)
    return jnp.sum(a1), jnp.sum(a2), jnp.sum(a3), (off + 15) >> 4


def _count_pass(src, nv_src, cands):
    c1v, c2v, c3v = cands
    z = jnp.zeros((16,), jnp.int32)

    def body(j, accs):
        for k in range(4):
            x = src[pl.ds((j * 4 + k) * 16, 16)]
            accs = _count3(x, accs, c1v, c2v, c3v)
        return accs

    n4 = (nv_src + 3) >> 2
    a1, a2, a3 = lax.fori_loop(0, n4, body, (z, z, z))
    return jnp.sum(a1), jnp.sum(a2), jnp.sum(a3)


def _sc_select_body(
    inputs_hbm, targets_hbm, out_hbm, row_a, row_b, key_v, buf_b, buf_c,
    tgt_v, out_v, sem_a, sem_b
):
    wid = lax.axis_index("s") * 2 + lax.axis_index("c")
    base = wid * ROWS_PER_W
    pltpu.sync_copy(targets_hbm.at[pl.ds(base, ROWS_PER_W)], tgt_v)
    lane = lax.iota(jnp.int32, 16)
    l0 = lane == 0

    def process_row(r, row_v):
        tb = plsc.load_gather(tgt_v, [jnp.full((16,), r, jnp.int32)])

        cands0 = _cand_vecs(jnp.int32(0), 30)
        c1v, c2v, c3v = cands0
        z = jnp.zeros((16,), jnp.int32)

        def key_body(j, accs):
            for k in range(4):
                jj = j * 4 + k
                x = row_v[pl.ds(jj * 16, 16)]
                b = plsc.bitcast(x, jnp.int32)
                s = jnp.where(b >= 0, b, b ^ _LOW31)
                key_v[pl.ds(jj * 16, 16)] = s
                accs = _count3(s, accs, c1v, c2v, c3v)
            return accs

        a1, a2, a3 = lax.fori_loop(0, VREGS // 4, key_body, (z, z, z))
        s_pos = plsc.load_gather(key_v, [tb])[0]
        plsc.store_scatter(
            key_v, [tb], jnp.full((16,), _MIN32, jnp.int32), mask=l0
        )
        n1 = jnp.sum(a1) - jnp.where(s_pos >= c1v[0], 1, 0)
        n2 = jnp.sum(a2) - jnp.where(s_pos >= c2v[0], 1, 0)
        n3 = jnp.sum(a3) - jnp.where(s_pos >= c3v[0], 1, 0)
        p_u, nh0 = _decide(jnp.int32(0), jnp.int32(0), jnp.int32(30), n1, n2, n3)

        n1, n2, n3, nv_b = _compact_count_pass(
            key_v, buf_b, jnp.int32(VREGS), p_u, jnp.int32(30), _cand_vecs(p_u, 28)
        )
        p_u, nh1 = _decide(p_u, nh0, jnp.int32(28), n1, n2, n3)

        n1, n2, n3, nv_c = _compact_count_pass(
            buf_b, buf_c, nv_b, p_u, jnp.int32(28), _cand_vecs(p_u, 26)
        )
        p_u, nh2 = _decide(
            p_u, nh1, jnp.int32(26), n1 + nh0, n2 + nh0, n3 + nh0
        )

        n1, n2, n3, nv_d = _compact_count_pass(
            buf_c, buf_b, nv_c, p_u, jnp.int32(26), _cand_vecs(p_u, 24)
        )
        p_u, nh3 = _decide(
            p_u, nh2, jnp.int32(24), n1 + nh1, n2 + nh1, n3 + nh1
        )

        def pass_body(q, p_u):
            shift = 30 - 2 * q
            n1, n2, n3 = _count_pass(buf_b, nv_d, _cand_vecs(p_u, shift))
            p_new, _ = _decide(
                p_u, jnp.int32(0), shift, n1 + nh2, n2 + nh2, n3 + nh2
            )
            return p_new

        p_u = lax.fori_loop(4, 16, pass_body, p_u)
        plsc.store_scatter(out_v, [jnp.full((16,), r, jnp.int32)],
                           jnp.full((16,), p_u, jnp.int32), mask=l0)

    pltpu.async_copy(inputs_hbm.at[base], row_a, sem_a)

    def grp_body(g, carry):
        for sub, (cur, nxt, cs, ns) in enumerate(
            ((row_a, row_b, sem_a, sem_b), (row_b, row_a, sem_b, sem_a))
        ):
            r = g * 2 + sub
            nr = (r + 1) & (ROWS_PER_W - 1)
            pltpu.async_copy(inputs_hbm.at[base + nr], nxt, ns)
            pltpu.make_async_copy(inputs_hbm.at[base + r], cur, cs).wait()
            process_row(r, cur)
        return carry

    lax.fori_loop(0, ROWS_PER_W // 2, grp_body, 0)
    pltpu.make_async_copy(inputs_hbm.at[base], row_a, sem_a).wait()
    pltpu.sync_copy(out_v, out_hbm.at[pl.ds(base, ROWS_PER_W)])


_sc_select = functools.partial(
    pl.kernel,
    out_type=jax.ShapeDtypeStruct((M,), jnp.int32),
    mesh=plsc.VectorSubcoreMesh(core_axis_name="c", subcore_axis_name="s"),
    scratch_types=[
        pltpu.VMEM((N,), jnp.float32),
        pltpu.VMEM((N,), jnp.float32),
        pltpu.VMEM((N,), jnp.int32),
        pltpu.VMEM((N + 128,), jnp.int32),
        pltpu.VMEM((N + 128,), jnp.int32),
        pltpu.VMEM((ROWS_PER_W,), jnp.int32),
        pltpu.VMEM((ROWS_PER_W,), jnp.int32),
        pltpu.SemaphoreType.DMA,
        pltpu.SemaphoreType.DMA,
    ],
    compiler_params=pltpu.CompilerParams(needs_layout_passes=False),
)(_sc_select_body)


def _softplus(x):
    return jnp.maximum(x, 0.0) + jnp.log1p(jnp.exp(-jnp.abs(x)))


def _finalize_body(x_ref, tgt_ref, tu_ref, out_ref):
    pid = pl.program_id(0)
    x = x_ref[...]
    tgt = tgt_ref[pl.ds(pid * BLOCK_M, BLOCK_M), :]
    t_u = tu_ref[pl.ds(pid * BLOCK_M, BLOCK_M), :]

    col = jax.lax.broadcasted_iota(jnp.int32, (BLOCK_M, N), 1)
    pos_mask = col == tgt
    neg_mask = jnp.logical_not(pos_mask)

    bits = jax.lax.bitcast_convert_type(x, jnp.int32)
    s = jnp.where(bits >= 0, bits, bits ^ _LOW31)

    t_s = t_u ^ _SIGN
    t_bits = jnp.where(t_s >= 0, t_s, t_s ^ _LOW31)
    t_f = jax.lax.bitcast_convert_type(t_bits, jnp.float32)

    gt = (s > t_s) & neg_mask
    c = jnp.sum(gt.astype(jnp.int32), axis=1, keepdims=True).astype(jnp.float32)
    sp = _softplus(x)
    sum_sp = jnp.sum(jnp.where(gt, sp, 0.0), axis=1, keepdims=True)
    l_neg = (sum_sp + (K - c) * _softplus(t_f)) * (1.0 / K)

    pos = jnp.sum(jnp.where(pos_mask, x, 0.0), axis=1, keepdims=True)
    per_row = _softplus(-pos) + l_neg

    @pl.when(pid == 0)
    def _():
        out_ref[...] = jnp.zeros((1, 1), jnp.float32)

    out_ref[...] += jnp.sum(per_row).reshape(1, 1) * (1.0 / M)


@jax.jit
def kernel(inputs, targets):
    tgt = targets.astype(jnp.int32)
    t_u = _sc_select(inputs, tgt)
    grid = M // BLOCK_M
    out = pl.pallas_call(
        _finalize_body,
        grid=(grid,),
        in_specs=[
            pl.BlockSpec((BLOCK_M, N), lambda i: (i, 0)),
            pl.BlockSpec((M, 1), lambda i: (0, 0)),
            pl.BlockSpec((M, 1), lambda i: (0, 0)),
        ],
        out_specs=pl.BlockSpec((1, 1), lambda i: (0, 0)),
        out_shape=jax.ShapeDtypeStruct((1, 1), jnp.float32),
        compiler_params=pltpu.CompilerParams(
            dimension_semantics=("arbitrary",),
        ),
    )(inputs, tgt.reshape(M, 1), t_u.reshape(M, 1))
    return out[0, 0]


# speedup vs baseline: 1.9896x; 1.3120x over previous
import functools

import jax
import jax.numpy as jnp
import numpy as np
from jax import lax
from jax.experimental import pallas as pl
from jax.experimental.pallas import tpu as pltpu
from jax.experimental.pallas import tpu_sc as plsc

M = 1024
N = 8192
K = N // 2 - 1
BLOCK_M = 128

NW = 32
ROWS_PER_W = M // NW
VREGS = N // 16

_SIGN = np.int32(np.uint32(0x80000000))
_LOW31 = np.int32(0x7FFFFFFF)


_MIN32 = np.int32(np.uint32(0x80000000))


def _flag(n):
    return jnp.where(n >= K, 1, 0)


def _decide(p_u, nh, shift, n1, n2, n3):
    bits = _flag(n1) + _flag(n2) + _flag(n3)
    p_new = p_u | lax.shift_left(bits, shift)
    nh_new = jnp.where(
        bits == 0, n1, jnp.where(bits == 1, n2, jnp.where(bits == 2, n3, nh))
    )
    return p_new, nh_new


def _cand_vecs(p_u, shift):
    c1 = (p_u | lax.shift_left(jnp.int32(1), shift)) ^ _SIGN
    c2 = (p_u | lax.shift_left(jnp.int32(2), shift)) ^ _SIGN
    c3 = (p_u | lax.shift_left(jnp.int32(3), shift)) ^ _SIGN
    return (
        jnp.full((16,), c1, jnp.int32),
        jnp.full((16,), c2, jnp.int32),
        jnp.full((16,), c3, jnp.int32),
    )


def _count3(x, accs, c1v, c2v, c3v):
    a1, a2, a3 = accs
    a1 = a1 + jnp.where(x >= c1v, 1, 0)
    a2 = a2 + jnp.where(x >= c2v, 1, 0)
    a3 = a3 + jnp.where(x >= c3v, 1, 0)
    return (a1, a2, a3)


def _compact_pass(src, dst, nv_src, p_u, shift_prev):
    lo_v = jnp.full((16,), p_u ^ _SIGN, jnp.int32)
    hi_u = p_u + lax.shift_left(jnp.int32(1), shift_prev)
    hi_v = jnp.full((16,), hi_u ^ _SIGN, jnp.int32)
    hz_v = jnp.full((16,), hi_u, jnp.int32) == 0
    z = jnp.zeros((16,), jnp.int32)
    lane = lax.iota(jnp.int32, 16)
    idx15 = jnp.full((16,), 15, jnp.int32)

    def body(j, off_v):
        for k in range(4):
            x = src[pl.ds((j * 4 + k) * 16, 16)]
            m_in = (x >= lo_v) & ((x < hi_v) | hz_v)
            cum = plsc.cumsum(jnp.where(m_in, 1, 0))
            plsc.store_scatter(dst, [off_v + (cum - 1)], x, mask=m_in)
            off_v = off_v + cum.at[idx15].get(mode="promise_in_bounds")
        return off_v

    n4 = (nv_src + 3) >> 2
    off_v = lax.fori_loop(0, n4, body, z)
    off = off_v[0]
    minv = jnp.full((16,), _MIN32, jnp.int32)
    ones = lane >= 0
    for k in range(4):
        plsc.store_scatter(dst, [off + k * 16 + lane], minv, mask=ones)
    return (off + 15) >> 4


def _compact_count_pass(src, dst, nv_src, p_u, shift_prev, cands):
    c1v, c2v, c3v = cands
    lo_v = jnp.full((16,), p_u ^ _SIGN, jnp.int32)
    hi_u = p_u + lax.shift_left(jnp.int32(1), shift_prev)
    hi_v = jnp.full((16,), hi_u ^ _SIGN, jnp.int32)
    hz_v = jnp.full((16,), hi_u, jnp.int32) == 0
    z = jnp.zeros((16,), jnp.int32)
    lane = lax.iota(jnp.int32, 16)

    idx15 = jnp.full((16,), 15, jnp.int32)

    def body(j, carry):
        a1, a2, a3, off_v = carry
        xs, cums = [], []
        for k in range(4):
            x = src[pl.ds((j * 4 + k) * 16, 16)]
            m_in = (x >= lo_v) & ((x < hi_v) | hz_v)
            xs.append((x, m_in))
            cums.append(plsc.cumsum(jnp.where(m_in, 1, 0)))
        for k in range(4):
            x, m_in = xs[k]
            plsc.store_scatter(dst, [off_v + (cums[k] - 1)], x, mask=m_in)
            off_v = off_v + cums[k].at[idx15].get(mode="promise_in_bounds")
            a1, a2, a3 = _count3(x, (a1, a2, a3), c1v, c2v, c3v)
        return (a1, a2, a3, off_v)

    n4 = (nv_src + 3) >> 2
    a1, a2, a3, off_v = lax.fori_loop(0, n4, body, (z, z, z, z))
    off = off_v[0]
    minv = jnp.full((16,), _MIN32, jnp.int32)
    ones = lane >= 0
    for k in range(4):
        plsc.store_scatter(dst, [off + k * 16 + lane], minv, mask=ones)
    return jnp.sum(a1), jnp.sum(a2), jnp.sum(a3), (off + 15) >> 4


def _count_pass(src, nv_src, cands):
    c1v, c2v, c3v = cands
    z = jnp.zeros((16,), jnp.int32)

    def body(j, accs):
        for k in range(4):
            x = src[pl.ds((j * 4 + k) * 16, 16)]
            accs = _count3(x, accs, c1v, c2v, c3v)
        return accs

    n4 = (nv_src + 3) >> 2
    a1, a2, a3 = lax.fori_loop(0, n4, body, (z, z, z))
    return jnp.sum(a1), jnp.sum(a2), jnp.sum(a3)


def _sc_select_body(
    inputs_hbm, targets_hbm, out_hbm, row_a, row_b, key_v, buf_b, buf_c,
    tgt_v, out_v, sem_a, sem_b
):
    wid = lax.axis_index("s") * 2 + lax.axis_index("c")
    base = wid * ROWS_PER_W
    pltpu.sync_copy(targets_hbm.at[pl.ds(base, ROWS_PER_W)], tgt_v)
    lane = lax.iota(jnp.int32, 16)
    l0 = lane == 0

    def process_row(r, row_v):
        tb = plsc.load_gather(tgt_v, [jnp.full((16,), r, jnp.int32)])

        cands0 = _cand_vecs(jnp.int32(0), 30)
        c1v, c2v, c3v = cands0
        z = jnp.zeros((16,), jnp.int32)

        def key_body(j, accs):
            for k in range(4):
                jj = j * 4 + k
                x = row_v[pl.ds(jj * 16, 16)]
                b = plsc.bitcast(x, jnp.int32)
                s = jnp.where(b >= 0, b, b ^ _LOW31)
                key_v[pl.ds(jj * 16, 16)] = s
                accs = _count3(s, accs, c1v, c2v, c3v)
            return accs

        a1, a2, a3 = lax.fori_loop(0, VREGS // 4, key_body, (z, z, z))
        s_pos = plsc.load_gather(key_v, [tb])[0]
        plsc.store_scatter(
            key_v, [tb], jnp.full((16,), _MIN32, jnp.int32), mask=l0
        )
        n1 = jnp.sum(a1) - jnp.where(s_pos >= c1v[0], 1, 0)
        n2 = jnp.sum(a2) - jnp.where(s_pos >= c2v[0], 1, 0)
        n3 = jnp.sum(a3) - jnp.where(s_pos >= c3v[0], 1, 0)
        p_u, nh0 = _decide(jnp.int32(0), jnp.int32(0), jnp.int32(30), n1, n2, n3)

        n1, n2, n3, nv_b = _compact_count_pass(
            key_v, buf_b, jnp.int32(VREGS), p_u, jnp.int32(30), _cand_vecs(p_u, 28)
        )
        p_u, nh1 = _decide(p_u, nh0, jnp.int32(28), n1, n2, n3)

        n1, n2, n3, nv_c = _compact_count_pass(
            buf_b, buf_c, nv_b, p_u, jnp.int32(28), _cand_vecs(p_u, 26)
        )
        p_u, nh2 = _decide(
            p_u, nh1, jnp.int32(26), n1 + nh0, n2 + nh0, n3 + nh0
        )

        n1, n2, n3, nv_d = _compact_count_pass(
            buf_c, buf_b, nv_c, p_u, jnp.int32(26), _cand_vecs(p_u, 24)
        )
        p_u, nh3 = _decide(
            p_u, nh2, jnp.int32(24), n1 + nh1, n2 + nh1, n3 + nh1
        )

        n1, n2, n3, nv_e = _compact_count_pass(
            buf_b, buf_c, nv_d, p_u, jnp.int32(24), _cand_vecs(p_u, 22)
        )
        p_u, nh4 = _decide(
            p_u, nh3, jnp.int32(22), n1 + nh2, n2 + nh2, n3 + nh2
        )

        def pass_body(q, p_u):
            shift = 30 - 2 * q
            n1, n2, n3 = _count_pass(buf_c, nv_e, _cand_vecs(p_u, shift))
            p_new, _ = _decide(
                p_u, jnp.int32(0), shift, n1 + nh3, n2 + nh3, n3 + nh3
            )
            return p_new

        p_u = lax.fori_loop(5, 16, pass_body, p_u)
        plsc.store_scatter(out_v, [jnp.full((16,), r, jnp.int32)],
                           jnp.full((16,), p_u, jnp.int32), mask=l0)

    pltpu.async_copy(inputs_hbm.at[base], row_a, sem_a)

    def grp_body(g, carry):
        for sub, (cur, nxt, cs, ns) in enumerate(
            ((row_a, row_b, sem_a, sem_b), (row_b, row_a, sem_b, sem_a))
        ):
            r = g * 2 + sub
            nr = (r + 1) & (ROWS_PER_W - 1)
            pltpu.async_copy(inputs_hbm.at[base + nr], nxt, ns)
            pltpu.make_async_copy(inputs_hbm.at[base + r], cur, cs).wait()
            process_row(r, cur)
        return carry

    lax.fori_loop(0, ROWS_PER_W // 2, grp_body, 0)
    pltpu.make_async_copy(inputs_hbm.at[base], row_a, sem_a).wait()
    pltpu.sync_copy(out_v, out_hbm.at[pl.ds(base, ROWS_PER_W)])


_sc_select = functools.partial(
    pl.kernel,
    out_type=jax.ShapeDtypeStruct((M,), jnp.int32),
    mesh=plsc.VectorSubcoreMesh(core_axis_name="c", subcore_axis_name="s"),
    scratch_types=[
        pltpu.VMEM((N,), jnp.float32),
        pltpu.VMEM((N,), jnp.float32),
        pltpu.VMEM((N,), jnp.int32),
        pltpu.VMEM((N + 128,), jnp.int32),
        pltpu.VMEM((N + 128,), jnp.int32),
        pltpu.VMEM((ROWS_PER_W,), jnp.int32),
        pltpu.VMEM((ROWS_PER_W,), jnp.int32),
        pltpu.SemaphoreType.DMA,
        pltpu.SemaphoreType.DMA,
    ],
    compiler_params=pltpu.CompilerParams(needs_layout_passes=False),
)(_sc_select_body)


def _softplus(x):
    return jnp.maximum(x, 0.0) + jnp.log1p(jnp.exp(-jnp.abs(x)))


def _finalize_body(x_ref, tgt_ref, tu_ref, out_ref):
    pid = pl.program_id(0)
    x = x_ref[...]
    tgt = tgt_ref[pl.ds(pid * BLOCK_M, BLOCK_M), :]
    t_u = tu_ref[pl.ds(pid * BLOCK_M, BLOCK_M), :]

    col = jax.lax.broadcasted_iota(jnp.int32, (BLOCK_M, N), 1)
    pos_mask = col == tgt
    neg_mask = jnp.logical_not(pos_mask)

    bits = jax.lax.bitcast_convert_type(x, jnp.int32)
    s = jnp.where(bits >= 0, bits, bits ^ _LOW31)

    t_s = t_u ^ _SIGN
    t_bits = jnp.where(t_s >= 0, t_s, t_s ^ _LOW31)
    t_f = jax.lax.bitcast_convert_type(t_bits, jnp.float32)

    gt = (s > t_s) & neg_mask
    c = jnp.sum(gt.astype(jnp.int32), axis=1, keepdims=True).astype(jnp.float32)
    sp = _softplus(x)
    sum_sp = jnp.sum(jnp.where(gt, sp, 0.0), axis=1, keepdims=True)
    l_neg = (sum_sp + (K - c) * _softplus(t_f)) * (1.0 / K)

    pos = jnp.sum(jnp.where(pos_mask, x, 0.0), axis=1, keepdims=True)
    per_row = _softplus(-pos) + l_neg

    @pl.when(pid == 0)
    def _():
        out_ref[...] = jnp.zeros((1, 1), jnp.float32)

    out_ref[...] += jnp.sum(per_row).reshape(1, 1) * (1.0 / M)


@jax.jit
def kernel(inputs, targets):
    tgt = targets.astype(jnp.int32)
    t_u = _sc_select(inputs, tgt)
    grid = M // BLOCK_M
    out = pl.pallas_call(
        _finalize_body,
        grid=(grid,),
        in_specs=[
            pl.BlockSpec((BLOCK_M, N), lambda i: (i, 0)),
            pl.BlockSpec((M, 1), lambda i: (0, 0)),
            pl.BlockSpec((M, 1), lambda i: (0, 0)),
        ],
        out_specs=pl.BlockSpec((1, 1), lambda i: (0, 0)),
        out_shape=jax.ShapeDtypeStruct((1, 1), jnp.float32),
        compiler_params=pltpu.CompilerParams(
            dimension_semantics=("arbitrary",),
        ),
    )(inputs, tgt.reshape(M, 1), t_u.reshape(M, 1))
    return out[0, 0]


# speedup vs baseline: 1.9976x; 1.0040x over previous
import functools

import jax
import jax.numpy as jnp
import numpy as np
from jax import lax
from jax.experimental import pallas as pl
from jax.experimental.pallas import tpu as pltpu
from jax.experimental.pallas import tpu_sc as plsc

M = 1024
N = 8192
K = N // 2 - 1
BLOCK_M = 128

NW = 32
ROWS_PER_W = M // NW
VREGS = N // 16

_SIGN = np.int32(np.uint32(0x80000000))
_LOW31 = np.int32(0x7FFFFFFF)


_MIN32 = np.int32(np.uint32(0x80000000))


def _flag(n):
    return jnp.where(n >= K, 1, 0)


def _decide(p_u, nh, shift, n1, n2, n3):
    bits = _flag(n1) + _flag(n2) + _flag(n3)
    p_new = p_u | lax.shift_left(bits, shift)
    nh_new = jnp.where(
        bits == 0, n1, jnp.where(bits == 1, n2, jnp.where(bits == 2, n3, nh))
    )
    return p_new, nh_new


def _cand_vecs(p_u, shift):
    c1 = (p_u | lax.shift_left(jnp.int32(1), shift)) ^ _SIGN
    c2 = (p_u | lax.shift_left(jnp.int32(2), shift)) ^ _SIGN
    c3 = (p_u | lax.shift_left(jnp.int32(3), shift)) ^ _SIGN
    return (
        jnp.full((16,), c1, jnp.int32),
        jnp.full((16,), c2, jnp.int32),
        jnp.full((16,), c3, jnp.int32),
    )


def _count3(x, accs, c1v, c2v, c3v):
    a1, a2, a3 = accs
    a1 = a1 + jnp.where(x >= c1v, 1, 0)
    a2 = a2 + jnp.where(x >= c2v, 1, 0)
    a3 = a3 + jnp.where(x >= c3v, 1, 0)
    return (a1, a2, a3)


def _compact_pass(src, dst, nv_src, p_u, shift_prev):
    lo_v = jnp.full((16,), p_u ^ _SIGN, jnp.int32)
    hi_u = p_u + lax.shift_left(jnp.int32(1), shift_prev)
    hi_v = jnp.full((16,), hi_u ^ _SIGN, jnp.int32)
    hz_v = jnp.full((16,), hi_u, jnp.int32) == 0
    z = jnp.zeros((16,), jnp.int32)
    lane = lax.iota(jnp.int32, 16)
    idx15 = jnp.full((16,), 15, jnp.int32)

    def body(j, off_v):
        for k in range(4):
            x = src[pl.ds((j * 4 + k) * 16, 16)]
            m_in = (x >= lo_v) & ((x < hi_v) | hz_v)
            cum = plsc.cumsum(jnp.where(m_in, 1, 0))
            plsc.store_scatter(dst, [off_v + (cum - 1)], x, mask=m_in)
            off_v = off_v + cum.at[idx15].get(mode="promise_in_bounds")
        return off_v

    n4 = (nv_src + 3) >> 2
    off_v = lax.fori_loop(0, n4, body, z)
    off = off_v[0]
    minv = jnp.full((16,), _MIN32, jnp.int32)
    ones = lane >= 0
    for k in range(4):
        plsc.store_scatter(dst, [off + k * 16 + lane], minv, mask=ones)
    return (off + 15) >> 4


def _compact_count_pass(src, dst, nv_src, p_u, shift_prev, cands):
    c1v, c2v, c3v = cands
    lo_v = jnp.full((16,), p_u ^ _SIGN, jnp.int32)
    hi_u = p_u + lax.shift_left(jnp.int32(1), shift_prev)
    hi_v = jnp.full((16,), hi_u ^ _SIGN, jnp.int32)
    hz_v = jnp.full((16,), hi_u, jnp.int32) == 0
    z = jnp.zeros((16,), jnp.int32)
    lane = lax.iota(jnp.int32, 16)

    idx15 = jnp.full((16,), 15, jnp.int32)

    def body(j, carry):
        a1, a2, a3, off_v = carry
        xs, cums = [], []
        for k in range(4):
            x = src[pl.ds((j * 4 + k) * 16, 16)]
            m_in = (x >= lo_v) & ((x < hi_v) | hz_v)
            xs.append((x, m_in))
            cums.append(plsc.cumsum(jnp.where(m_in, 1, 0)))
        for k in range(4):
            x, m_in = xs[k]
            plsc.store_scatter(dst, [off_v + (cums[k] - 1)], x, mask=m_in)
            off_v = off_v + cums[k].at[idx15].get(mode="promise_in_bounds")
            a1, a2, a3 = _count3(x, (a1, a2, a3), c1v, c2v, c3v)
        return (a1, a2, a3, off_v)

    n4 = (nv_src + 3) >> 2
    a1, a2, a3, off_v = lax.fori_loop(0, n4, body, (z, z, z, z))
    off = off_v[0]
    minv = jnp.full((16,), _MIN32, jnp.int32)
    ones = lane >= 0
    for k in range(4):
        plsc.store_scatter(dst, [off + k * 16 + lane], minv, mask=ones)
    return jnp.sum(a1), jnp.sum(a2), jnp.sum(a3), (off + 15) >> 4


def _count_pass(src, nv_src, cands):
    c1v, c2v, c3v = cands
    z = jnp.zeros((16,), jnp.int32)

    def body(j, accs):
        a1, a2, a3 = accs
        for k in range(4):
            x = src[pl.ds((j * 4 + k) * 16, 16)]
            a1 = a1 + plsc.all_reduce_population_count(x >= c1v)
            a2 = a2 + plsc.all_reduce_population_count(x >= c2v)
            a3 = a3 + plsc.all_reduce_population_count(x >= c3v)
        return (a1, a2, a3)

    n4 = (nv_src + 3) >> 2
    a1, a2, a3 = lax.fori_loop(0, n4, body, (z, z, z))
    return a1[0], a2[0], a3[0]


def _sc_select_body(
    inputs_hbm, targets_hbm, out_hbm, row_a, row_b, key_v, buf_b, buf_c,
    tgt_v, out_v, sem_a, sem_b
):
    wid = lax.axis_index("s") * 2 + lax.axis_index("c")
    base = wid * ROWS_PER_W
    pltpu.sync_copy(targets_hbm.at[pl.ds(base, ROWS_PER_W)], tgt_v)
    lane = lax.iota(jnp.int32, 16)
    l0 = lane == 0

    def process_row(r, row_v):
        tb = plsc.load_gather(tgt_v, [jnp.full((16,), r, jnp.int32)])

        cands0 = _cand_vecs(jnp.int32(0), 30)
        c1v, c2v, c3v = cands0
        z = jnp.zeros((16,), jnp.int32)

        def key_body(j, accs):
            for k in range(4):
                jj = j * 4 + k
                x = row_v[pl.ds(jj * 16, 16)]
                b = plsc.bitcast(x, jnp.int32)
                s = jnp.where(b >= 0, b, b ^ _LOW31)
                key_v[pl.ds(jj * 16, 16)] = s
                accs = _count3(s, accs, c1v, c2v, c3v)
            return accs

        a1, a2, a3 = lax.fori_loop(0, VREGS // 4, key_body, (z, z, z))
        s_pos = plsc.load_gather(key_v, [tb])[0]
        plsc.store_scatter(
            key_v, [tb], jnp.full((16,), _MIN32, jnp.int32), mask=l0
        )
        n1 = jnp.sum(a1) - jnp.where(s_pos >= c1v[0], 1, 0)
        n2 = jnp.sum(a2) - jnp.where(s_pos >= c2v[0], 1, 0)
        n3 = jnp.sum(a3) - jnp.where(s_pos >= c3v[0], 1, 0)
        p_u, nh0 = _decide(jnp.int32(0), jnp.int32(0), jnp.int32(30), n1, n2, n3)

        n1, n2, n3, nv_b = _compact_count_pass(
            key_v, buf_b, jnp.int32(VREGS), p_u, jnp.int32(30), _cand_vecs(p_u, 28)
        )
        p_u, nh1 = _decide(p_u, nh0, jnp.int32(28), n1, n2, n3)

        n1, n2, n3, nv_c = _compact_count_pass(
            buf_b, buf_c, nv_b, p_u, jnp.int32(28), _cand_vecs(p_u, 26)
        )
        p_u, nh2 = _decide(
            p_u, nh1, jnp.int32(26), n1 + nh0, n2 + nh0, n3 + nh0
        )

        n1, n2, n3, nv_d = _compact_count_pass(
            buf_c, buf_b, nv_c, p_u, jnp.int32(26), _cand_vecs(p_u, 24)
        )
        p_u, nh3 = _decide(
            p_u, nh2, jnp.int32(24), n1 + nh1, n2 + nh1, n3 + nh1
        )

        n1, n2, n3, nv_e = _compact_count_pass(
            buf_b, buf_c, nv_d, p_u, jnp.int32(24), _cand_vecs(p_u, 22)
        )
        p_u, nh4 = _decide(
            p_u, nh3, jnp.int32(22), n1 + nh2, n2 + nh2, n3 + nh2
        )

        def pass_body(q, p_u):
            shift = 30 - 2 * q
            n1, n2, n3 = _count_pass(buf_c, nv_e, _cand_vecs(p_u, shift))
            p_new, _ = _decide(
                p_u, jnp.int32(0), shift, n1 + nh3, n2 + nh3, n3 + nh3
            )
            return p_new

        p_u = lax.fori_loop(5, 16, pass_body, p_u)
        plsc.store_scatter(out_v, [jnp.full((16,), r, jnp.int32)],
                           jnp.full((16,), p_u, jnp.int32), mask=l0)

    pltpu.async_copy(inputs_hbm.at[base], row_a, sem_a)

    def grp_body(g, carry):
        for sub, (cur, nxt, cs, ns) in enumerate(
            ((row_a, row_b, sem_a, sem_b), (row_b, row_a, sem_b, sem_a))
        ):
            r = g * 2 + sub
            nr = (r + 1) & (ROWS_PER_W - 1)
            pltpu.async_copy(inputs_hbm.at[base + nr], nxt, ns)
            pltpu.make_async_copy(inputs_hbm.at[base + r], cur, cs).wait()
            process_row(r, cur)
        return carry

    lax.fori_loop(0, ROWS_PER_W // 2, grp_body, 0)
    pltpu.make_async_copy(inputs_hbm.at[base], row_a, sem_a).wait()
    pltpu.sync_copy(out_v, out_hbm.at[pl.ds(base, ROWS_PER_W)])


_sc_select = functools.partial(
    pl.kernel,
    out_type=jax.ShapeDtypeStruct((M,), jnp.int32),
    mesh=plsc.VectorSubcoreMesh(core_axis_name="c", subcore_axis_name="s"),
    scratch_types=[
        pltpu.VMEM((N,), jnp.float32),
        pltpu.VMEM((N,), jnp.float32),
        pltpu.VMEM((N,), jnp.int32),
        pltpu.VMEM((N + 128,), jnp.int32),
        pltpu.VMEM((N + 128,), jnp.int32),
        pltpu.VMEM((ROWS_PER_W,), jnp.int32),
        pltpu.VMEM((ROWS_PER_W,), jnp.int32),
        pltpu.SemaphoreType.DMA,
        pltpu.SemaphoreType.DMA,
    ],
    compiler_params=pltpu.CompilerParams(needs_layout_passes=False),
)(_sc_select_body)


def _softplus(x):
    return jnp.maximum(x, 0.0) + jnp.log1p(jnp.exp(-jnp.abs(x)))


def _finalize_body(x_ref, tgt_ref, tu_ref, out_ref):
    pid = pl.program_id(0)
    x = x_ref[...]
    tgt = tgt_ref[pl.ds(pid * BLOCK_M, BLOCK_M), :]
    t_u = tu_ref[pl.ds(pid * BLOCK_M, BLOCK_M), :]

    col = jax.lax.broadcasted_iota(jnp.int32, (BLOCK_M, N), 1)
    pos_mask = col == tgt
    neg_mask = jnp.logical_not(pos_mask)

    bits = jax.lax.bitcast_convert_type(x, jnp.int32)
    s = jnp.where(bits >= 0, bits, bits ^ _LOW31)

    t_s = t_u ^ _SIGN
    t_bits = jnp.where(t_s >= 0, t_s, t_s ^ _LOW31)
    t_f = jax.lax.bitcast_convert_type(t_bits, jnp.float32)

    gt = (s > t_s) & neg_mask
    c = jnp.sum(gt.astype(jnp.int32), axis=1, keepdims=True).astype(jnp.float32)
    sp = _softplus(x)
    sum_sp = jnp.sum(jnp.where(gt, sp, 0.0), axis=1, keepdims=True)
    l_neg = (sum_sp + (K - c) * _softplus(t_f)) * (1.0 / K)

    pos = jnp.sum(jnp.where(pos_mask, x, 0.0), axis=1, keepdims=True)
    per_row = _softplus(-pos) + l_neg

    @pl.when(pid == 0)
    def _():
        out_ref[...] = jnp.zeros((1, 1), jnp.float32)

    out_ref[...] += jnp.sum(per_row).reshape(1, 1) * (1.0 / M)


@jax.jit
def kernel(inputs, targets):
    tgt = targets.astype(jnp.int32)
    t_u = _sc_select(inputs, tgt)
    grid = M // BLOCK_M
    out = pl.pallas_call(
        _finalize_body,
        grid=(grid,),
        in_specs=[
            pl.BlockSpec((BLOCK_M, N), lambda i: (i, 0)),
            pl.BlockSpec((M, 1), lambda i: (0, 0)),
            pl.BlockSpec((M, 1), lambda i: (0, 0)),
        ],
        out_specs=pl.BlockSpec((1, 1), lambda i: (0, 0)),
        out_shape=jax.ShapeDtypeStruct((1, 1), jnp.float32),
        compiler_params=pltpu.CompilerParams(
            dimension_semantics=("arbitrary",),
        ),
    )(inputs, tgt.reshape(M, 1), t_u.reshape(M, 1))
    return out[0, 0]


# speedup vs baseline: 1.9980x; 1.0002x over previous
import functools

import jax
import jax.numpy as jnp
import numpy as np
from jax import lax
from jax.experimental import pallas as pl
from jax.experimental.pallas import tpu as pltpu
from jax.experimental.pallas import tpu_sc as plsc

M = 1024
N = 8192
K = N // 2 - 1
BLOCK_M = 128

NW = 32
ROWS_PER_W = M // NW
VREGS = N // 16

_SIGN = np.int32(np.uint32(0x80000000))
_LOW31 = np.int32(0x7FFFFFFF)


_MIN32 = np.int32(np.uint32(0x80000000))


def _flag(n):
    return jnp.where(n >= K, 1, 0)


def _decide(p_u, nh, shift, n1, n2, n3):
    bits = _flag(n1) + _flag(n2) + _flag(n3)
    p_new = p_u | lax.shift_left(bits, shift)
    nh_new = jnp.where(
        bits == 0, n1, jnp.where(bits == 1, n2, jnp.where(bits == 2, n3, nh))
    )
    return p_new, nh_new


def _cand_vecs(p_u, shift):
    c1 = (p_u | lax.shift_left(jnp.int32(1), shift)) ^ _SIGN
    c2 = (p_u | lax.shift_left(jnp.int32(2), shift)) ^ _SIGN
    c3 = (p_u | lax.shift_left(jnp.int32(3), shift)) ^ _SIGN
    return (
        jnp.full((16,), c1, jnp.int32),
        jnp.full((16,), c2, jnp.int32),
        jnp.full((16,), c3, jnp.int32),
    )


def _count3(x, accs, c1v, c2v, c3v):
    a1, a2, a3 = accs
    a1 = a1 + jnp.where(x >= c1v, 1, 0)
    a2 = a2 + jnp.where(x >= c2v, 1, 0)
    a3 = a3 + jnp.where(x >= c3v, 1, 0)
    return (a1, a2, a3)


def _compact_count_pass(src, dst, nv_src, p_u, shift_prev, cands):
    c1v, c2v, c3v = cands
    lo_v = jnp.full((16,), p_u ^ _SIGN, jnp.int32)
    hi_u = p_u + lax.shift_left(jnp.int32(1), shift_prev)
    hi_v = jnp.full((16,), hi_u ^ _SIGN, jnp.int32)
    hz_v = jnp.full((16,), hi_u, jnp.int32) == 0
    z = jnp.zeros((16,), jnp.int32)
    lane = lax.iota(jnp.int32, 16)

    idx15 = jnp.full((16,), 15, jnp.int32)

    def body(j, carry):
        a1, a2, a3, off_v = carry
        xs, cums = [], []
        for k in range(4):
            x = src[pl.ds((j * 4 + k) * 16, 16)]
            m_in = (x >= lo_v) & ((x < hi_v) | hz_v)
            xs.append((x, m_in))
            cums.append(plsc.cumsum(jnp.where(m_in, 1, 0)))
        for k in range(4):
            x, m_in = xs[k]
            plsc.store_scatter(dst, [off_v + (cums[k] - 1)], x, mask=m_in)
            off_v = off_v + cums[k].at[idx15].get(mode="promise_in_bounds")
            a1, a2, a3 = _count3(x, (a1, a2, a3), c1v, c2v, c3v)
        return (a1, a2, a3, off_v)

    n4 = (nv_src + 3) >> 2
    a1, a2, a3, off_v = lax.fori_loop(0, n4, body, (z, z, z, z))
    off = off_v[0]
    minv = jnp.full((16,), _MIN32, jnp.int32)
    ones = lane >= 0
    for k in range(4):
        plsc.store_scatter(dst, [off + k * 16 + lane], minv, mask=ones)
    return jnp.sum(a1), jnp.sum(a2), jnp.sum(a3), (off + 15) >> 4


def _count_pass(src, nv_src, cands):
    c1v, c2v, c3v = cands
    z = jnp.zeros((16,), jnp.int32)

    def body(j, accs):
        a1, a2, a3 = accs
        for k in range(4):
            x = src[pl.ds((j * 4 + k) * 16, 16)]
            a1 = a1 + plsc.all_reduce_population_count(x >= c1v)
            a2 = a2 + plsc.all_reduce_population_count(x >= c2v)
            a3 = a3 + plsc.all_reduce_population_count(x >= c3v)
        return (a1, a2, a3)

    n4 = (nv_src + 3) >> 2
    a1, a2, a3 = lax.fori_loop(0, n4, body, (z, z, z))
    return a1[0], a2[0], a3[0]


def _sc_select_body(
    inputs_hbm, targets_hbm, out_hbm, row_a, row_b, key_v, buf_b, buf_c,
    tgt_v, out_v, sem_a, sem_b
):
    wid = lax.axis_index("s") * 2 + lax.axis_index("c")
    base = wid * ROWS_PER_W
    pltpu.sync_copy(targets_hbm.at[pl.ds(base, ROWS_PER_W)], tgt_v)
    lane = lax.iota(jnp.int32, 16)
    l0 = lane == 0

    def process_row(r, row_v):
        tb = plsc.load_gather(tgt_v, [jnp.full((16,), r, jnp.int32)])

        cands0 = _cand_vecs(jnp.int32(0), 30)
        c1v, c2v, c3v = cands0
        z = jnp.zeros((16,), jnp.int32)

        def key_body(j, accs):
            for k in range(4):
                jj = j * 4 + k
                x = row_v[pl.ds(jj * 16, 16)]
                b = plsc.bitcast(x, jnp.int32)
                s = jnp.where(b >= 0, b, b ^ _LOW31)
                key_v[pl.ds(jj * 16, 16)] = s
                accs = _count3(s, accs, c1v, c2v, c3v)
            return accs

        a1, a2, a3 = lax.fori_loop(0, VREGS // 4, key_body, (z, z, z))
        s_pos = plsc.load_gather(key_v, [tb])[0]
        plsc.store_scatter(
            key_v, [tb], jnp.full((16,), _MIN32, jnp.int32), mask=l0
        )
        n1 = jnp.sum(a1) - jnp.where(s_pos >= c1v[0], 1, 0)
        n2 = jnp.sum(a2) - jnp.where(s_pos >= c2v[0], 1, 0)
        n3 = jnp.sum(a3) - jnp.where(s_pos >= c3v[0], 1, 0)
        p_u, nh0 = _decide(jnp.int32(0), jnp.int32(0), jnp.int32(30), n1, n2, n3)

        n1, n2, n3, nv_b = _compact_count_pass(
            key_v, buf_b, jnp.int32(VREGS), p_u, jnp.int32(30), _cand_vecs(p_u, 28)
        )
        p_u, nh1 = _decide(p_u, nh0, jnp.int32(28), n1, n2, n3)

        n1, n2, n3, nv_c = _compact_count_pass(
            buf_b, buf_c, nv_b, p_u, jnp.int32(28), _cand_vecs(p_u, 26)
        )
        p_u, nh2 = _decide(
            p_u, nh1, jnp.int32(26), n1 + nh0, n2 + nh0, n3 + nh0
        )

        n1, n2, n3, nv_d = _compact_count_pass(
            buf_c, buf_b, nv_c, p_u, jnp.int32(26), _cand_vecs(p_u, 24)
        )
        p_u, nh3 = _decide(
            p_u, nh2, jnp.int32(24), n1 + nh1, n2 + nh1, n3 + nh1
        )

        n1, n2, n3, nv_e = _compact_count_pass(
            buf_b, buf_c, nv_d, p_u, jnp.int32(24), _cand_vecs(p_u, 22)
        )
        p_u, nh4 = _decide(
            p_u, nh3, jnp.int32(22), n1 + nh2, n2 + nh2, n3 + nh2
        )

        def pass_body(q, p_u):
            shift = 30 - 2 * q
            n1, n2, n3 = _count_pass(buf_c, nv_e, _cand_vecs(p_u, shift))
            p_new, _ = _decide(
                p_u, jnp.int32(0), shift, n1 + nh3, n2 + nh3, n3 + nh3
            )
            return p_new

        p_u = lax.fori_loop(5, 16, pass_body, p_u)
        plsc.store_scatter(out_v, [jnp.full((16,), r, jnp.int32)],
                           jnp.full((16,), p_u, jnp.int32), mask=l0)

    pltpu.async_copy(inputs_hbm.at[base], row_a, sem_a)

    def grp_body(g, carry):
        for sub, (cur, nxt, cs, ns) in enumerate(
            ((row_a, row_b, sem_a, sem_b), (row_b, row_a, sem_b, sem_a))
        ):
            r = g * 2 + sub
            nr = (r + 1) & (ROWS_PER_W - 1)
            pltpu.async_copy(inputs_hbm.at[base + nr], nxt, ns)
            pltpu.make_async_copy(inputs_hbm.at[base + r], cur, cs).wait()
            process_row(r, cur)
        return carry

    lax.fori_loop(0, ROWS_PER_W // 2, grp_body, 0)
    pltpu.make_async_copy(inputs_hbm.at[base], row_a, sem_a).wait()
    pltpu.sync_copy(out_v, out_hbm.at[pl.ds(base, ROWS_PER_W)])


_sc_select = functools.partial(
    pl.kernel,
    out_type=jax.ShapeDtypeStruct((M,), jnp.int32),
    mesh=plsc.VectorSubcoreMesh(core_axis_name="c", subcore_axis_name="s"),
    scratch_types=[
        pltpu.VMEM((N,), jnp.float32),
        pltpu.VMEM((N,), jnp.float32),
        pltpu.VMEM((N,), jnp.int32),
        pltpu.VMEM((N + 128,), jnp.int32),
        pltpu.VMEM((N + 128,), jnp.int32),
        pltpu.VMEM((ROWS_PER_W,), jnp.int32),
        pltpu.VMEM((ROWS_PER_W,), jnp.int32),
        pltpu.SemaphoreType.DMA,
        pltpu.SemaphoreType.DMA,
    ],
    compiler_params=pltpu.CompilerParams(needs_layout_passes=False),
)(_sc_select_body)


def _softplus(x):
    return jnp.maximum(x, 0.0) + jnp.log1p(jnp.exp(-jnp.abs(x)))


def _finalize_body(x_ref, tgt_ref, tu_ref, out_ref):
    pid = pl.program_id(0)
    x = x_ref[...]
    tgt = tgt_ref[pl.ds(pid * BLOCK_M, BLOCK_M), :]
    t_u = tu_ref[pl.ds(pid * BLOCK_M, BLOCK_M), :]

    col = jax.lax.broadcasted_iota(jnp.int32, (BLOCK_M, N), 1)
    pos_mask = col == tgt
    neg_mask = jnp.logical_not(pos_mask)

    bits = jax.lax.bitcast_convert_type(x, jnp.int32)
    s = jnp.where(bits >= 0, bits, bits ^ _LOW31)

    t_s = t_u ^ _SIGN
    t_bits = jnp.where(t_s >= 0, t_s, t_s ^ _LOW31)
    t_f = jax.lax.bitcast_convert_type(t_bits, jnp.float32)

    gt = (s > t_s) & neg_mask
    c = jnp.sum(gt.astype(jnp.int32), axis=1, keepdims=True).astype(jnp.float32)
    sp = _softplus(x)
    sum_sp = jnp.sum(jnp.where(gt, sp, 0.0), axis=1, keepdims=True)
    l_neg = (sum_sp + (K - c) * _softplus(t_f)) * (1.0 / K)

    pos = jnp.sum(jnp.where(pos_mask, x, 0.0), axis=1, keepdims=True)
    per_row = _softplus(-pos) + l_neg

    @pl.when(pid == 0)
    def _():
        out_ref[...] = jnp.zeros((1, 1), jnp.float32)

    out_ref[...] += jnp.sum(per_row).reshape(1, 1) * (1.0 / M)


@jax.jit
def kernel(inputs, targets):
    tgt = targets.astype(jnp.int32)
    t_u = _sc_select(inputs, tgt)
    grid = M // BLOCK_M
    out = pl.pallas_call(
        _finalize_body,
        grid=(grid,),
        in_specs=[
            pl.BlockSpec((BLOCK_M, N), lambda i: (i, 0)),
            pl.BlockSpec((M, 1), lambda i: (0, 0)),
            pl.BlockSpec((M, 1), lambda i: (0, 0)),
        ],
        out_specs=pl.BlockSpec((1, 1), lambda i: (0, 0)),
        out_shape=jax.ShapeDtypeStruct((1, 1), jnp.float32),
        compiler_params=pltpu.CompilerParams(
            dimension_semantics=("arbitrary",),
        ),
    )(inputs, tgt.reshape(M, 1), t_u.reshape(M, 1))
    return out[0, 0]
